# baseline (speedup 1.0000x reference)
import functools
from math import exp as np_exp

import jax
import jax.numpy as jnp
from jax import lax
from jax.experimental import pallas as pl
from jax.experimental.pallas import tpu as pltpu
from jax.experimental.pallas import tpu_sc as plsc

N = 10000
K = 32
CIN = 128
COUT = 128
NB_BASIS = 10
MAXR = 2.5
SIGMA = MAXR / NB_BASIS
INV2S2 = 1.0 / (2.0 * SIGMA * SIGMA)
E = N * K

_CH = 80
_L = 16


_RING = 4
_NSTAGE = 5
_ES = E // _NSTAGE
_EW = _ES // 32
_NCH = _EW // _CH
_NPASS = (_NCH + _RING - 1) // _RING


def _sc_gather_body(ft_hbm, xs_hbm, ys_hbm, zs_hbm, idx_hbm,
                    outf_hbm, outd_hbm,
                    idx_all, f0, f1, f2, f3, d0, d1, d2b, d3,
                    xs_v, ys_v, zs_v,
                    sf0, sf1, sf2, sf3, ss0, ss1, ss2, ss3, nc, ebase0):
    wid = lax.axis_index("s") * nc + lax.axis_index("c")
    wbase = wid * _EW
    fb = [f0, f1, f2, f3]
    db = [d0, d1, d2b, d3]
    sf = [sf0, sf1, sf2, sf3]
    ss = [ss0, ss1, ss2, ss3]

    pltpu.sync_copy(idx_hbm.at[pl.ds(pl.multiple_of(wbase, 8), _EW)], idx_all)
    pltpu.sync_copy(xs_hbm, xs_v)
    pltpu.sync_copy(ys_hbm, ys_v)
    pltpu.sync_copy(zs_hbm, zs_v)

    def idx_slice(c):
        return idx_all.at[pl.ds(c * _CH, _CH)]

    def ebase(c):
        return pl.multiple_of(wbase + c * _CH, 8)

    def stores_wait(b):
        base0 = pl.multiple_of(wbase, 8)
        pltpu.make_async_copy(fb[b], outf_hbm.at[pl.ds(base0, _CH)],
                              ss[b]).wait()
        pltpu.make_async_copy(db[b], outd_hbm.at[pl.ds(base0, _CH)],
                              ss[b]).wait()

    def body(p, carry):
        for b in range(_RING):
            c = _RING * p + b

            @pl.when(c < _NCH)
            def _():
                @pl.when(c >= _RING)
                def _():
                    stores_wait(b)
                pltpu.async_copy(ft_hbm.at[idx_slice(c)], fb[b], sf[b])

        for b in range(_RING):
            c = _RING * p + b

            @pl.when(c < _NCH)
            def _():
                for g in range(_CH // _L):
                    off = c * _CH + g * _L
                    nbr = idx_all[pl.ds(off, _L)]
                    own = lax.shift_right_logical(
                        ebase0 + wbase + off
                        + jnp.arange(_L, dtype=jnp.int32), 5)
                    dx = (plsc.load_gather(xs_v, [nbr])
                          - plsc.load_gather(xs_v, [own]))
                    dy = (plsc.load_gather(ys_v, [nbr])
                          - plsc.load_gather(ys_v, [own]))
                    dz = (plsc.load_gather(zs_v, [nbr])
                          - plsc.load_gather(zs_v, [own]))
                    db[b][pl.ds(g * _L, _L)] = dx * dx + dy * dy + dz * dz
                pltpu.make_async_copy(ft_hbm.at[idx_slice(c)], fb[b],
                                      sf[b]).wait()
                base = ebase(c)
                pltpu.async_copy(fb[b], outf_hbm.at[pl.ds(base, _CH)], ss[b])
                pltpu.async_copy(db[b], outd_hbm.at[pl.ds(base, _CH)], ss[b])
        return carry

    lax.fori_loop(0, _NPASS, body, 0)
    for b in range(_RING):
        stores_wait(b)


def _sc_gather(ft, xs, ys, zs, idx, ebase0):
    info = plsc.get_sparse_core_info()
    nc = info.num_cores
    mesh = plsc.VectorSubcoreMesh(core_axis_name="c", subcore_axis_name="s")
    fn = functools.partial(
        pl.kernel,
        mesh=mesh,
        out_type=(
            jax.ShapeDtypeStruct((_ES, CIN), jnp.float32),
            jax.ShapeDtypeStruct((_ES,), jnp.float32),
        ),
        scratch_types=(
            [pltpu.VMEM((_EW,), jnp.int32)]
            + [pltpu.VMEM((_CH, CIN), jnp.float32)] * _RING
            + [pltpu.VMEM((_CH,), jnp.float32)] * _RING
            + [pltpu.VMEM((N,), jnp.float32)] * 3
            + [pltpu.SemaphoreType.DMA] * (2 * _RING)
        ),
        compiler_params=pltpu.CompilerParams(needs_layout_passes=False),
    )(functools.partial(_sc_gather_body, nc=nc, ebase0=ebase0))
    return fn(ft, xs, ys, zs, idx)


_NBLK = 200
_GN = 8
_CON = _GN * K
_G = _NBLK // _GN
_ROWS = _GN * NB_BASIS
_DELTA = MAXR / (NB_BASIS - 1)
_C5 = 5.0 * _DELTA


def _tc_body(gf_ref, d2_ref, rm_ref, wbig_ref, out_ref):
    d2 = d2_ref[0]
    d = jnp.minimum(jnp.sqrt(d2 + 1e-12), 6.0)
    rm = rm_ref[0]
    a_lo = jnp.exp(d * d * (-INV2S2)) * rm
    dh = d - _C5
    a_hi = jnp.exp(dh * dh * (-INV2S2)) * rm
    u = jnp.exp(d * (2.0 * INV2S2 * _DELTA))

    rbs = []
    e = a_lo
    rbs.append(e)
    for m in range(1, 5):
        e = e * u
        rbs.append(e * float(np_exp(-INV2S2 * (m * _DELTA) ** 2)))
    e = a_hi
    rbs.append(e)
    for m in range(1, 5):
        e = e * u
        rbs.append(e * float(np_exp(-INV2S2 * _DELTA * _DELTA
                                    * (10 * m + m * m))))

    rbstack = jnp.stack(rbs, axis=1)
    tiled = jnp.broadcast_to(
        rbstack[:, None, :, :], (_G, _GN, NB_BASIS, _CON)
    ).reshape(_G, _ROWS, _CON)
    ri = lax.broadcasted_iota(jnp.int32, (_ROWS, _CON), 0)
    ci = lax.broadcasted_iota(jnp.int32, (_ROWS, _CON), 1)
    maskf = (ri // NB_BASIS == ci // K).astype(jnp.float32)
    lhs = (tiled * maskf[None]).astype(jnp.bfloat16)

    rhs = gf_ref[...].astype(jnp.bfloat16)
    s = lax.dot_general(
        lhs, rhs,
        dimension_numbers=(((2,), (1,)), ((0,), (0,))),
        preferred_element_type=jnp.float32)
    s_flat = s.reshape(_NBLK, NB_BASIS * CIN).astype(jnp.bfloat16)
    out_ref[...] = lax.dot_general(
        s_flat, wbig_ref[...],
        dimension_numbers=(((1,), (0,)), ((), ())),
        preferred_element_type=jnp.float32)


_NS = N // _NSTAGE


def _tc_compute(gf3, d2g, rmg, wbig):
    grid = (_NS // _NBLK,)
    return pl.pallas_call(
        _tc_body,
        grid=grid,
        in_specs=[
            pl.BlockSpec((_G, _CON, CIN), lambda i: (i, 0, 0)),
            pl.BlockSpec((1, _G, _CON), lambda i: (i, 0, 0)),
            pl.BlockSpec((1, _G, _CON), lambda i: (i, 0, 0)),
            pl.BlockSpec((NB_BASIS * CIN, COUT), lambda i: (0, 0)),
        ],
        out_specs=pl.BlockSpec((_NBLK, COUT), lambda i: (i, 0)),
        out_shape=jax.ShapeDtypeStruct((_NS, COUT), jnp.float32),
        compiler_params=pltpu.CompilerParams(
            dimension_semantics=("arbitrary",)),
    )(gf3, d2g, rmg, wbig)


def kernel(features, geometry, neighbors, rel_mask, W):
    ft = features.T
    xs = geometry[:, 0]
    ys = geometry[:, 1]
    zs = geometry[:, 2]
    idx = neighbors.reshape(-1).astype(jnp.int32)
    wbig = jnp.transpose(W, (0, 2, 1)).reshape(
        NB_BASIS * CIN, COUT).astype(jnp.bfloat16)
    rmg = rel_mask.reshape(N // _NBLK, _G, _CON)
    outs = []
    for s in range(_NSTAGE):
        gf, d2 = _sc_gather(ft, xs, ys, zs,
                            lax.slice(idx, (s * _ES,), ((s + 1) * _ES,)),
                            s * _ES)
        nb0 = s * (_NS // _NBLK)
        outs.append(_tc_compute(
            gf.reshape(_ES // _CON, _CON, CIN),
            d2.reshape(_NS // _NBLK, _G, _CON),
            lax.slice(rmg, (nb0, 0, 0),
                      (nb0 + _NS // _NBLK, _G, _CON)), wbig))
    return jnp.concatenate(outs, axis=0).T


# speedup vs baseline: 1.2919x; 1.2919x over previous
import functools
from math import exp as np_exp

import jax
import jax.numpy as jnp
from jax import lax
from jax.experimental import pallas as pl
from jax.experimental.pallas import tpu as pltpu
from jax.experimental.pallas import tpu_sc as plsc

N = 10000
K = 32
CIN = 128
COUT = 128
NB_BASIS = 10
MAXR = 2.5
SIGMA = MAXR / NB_BASIS
INV2S2 = 1.0 / (2.0 * SIGMA * SIGMA)
E = N * K

_CH = 80
_L = 16


_RING = 4
_NSTAGE = 1
_ES = E // _NSTAGE
_EW = _ES // 32
_NCH = _EW // _CH
_NPASS = (_NCH + _RING - 1) // _RING


def _sc_gather_body(ft_hbm, xs_hbm, ys_hbm, zs_hbm, idx_hbm,
                    outf_hbm, outd_hbm,
                    idx_all, f0, f1, f2, f3, d0, d1, d2b, d3,
                    xs_v, ys_v, zs_v,
                    sf0, sf1, sf2, sf3, ss0, ss1, ss2, ss3, nc, ebase0):
    wid = lax.axis_index("s") * nc + lax.axis_index("c")
    wbase = wid * _EW
    fb = [f0, f1, f2, f3]
    db = [d0, d1, d2b, d3]
    sf = [sf0, sf1, sf2, sf3]
    ss = [ss0, ss1, ss2, ss3]

    pltpu.sync_copy(idx_hbm.at[pl.ds(pl.multiple_of(wbase, 8), _EW)], idx_all)
    pltpu.sync_copy(xs_hbm, xs_v)
    pltpu.sync_copy(ys_hbm, ys_v)
    pltpu.sync_copy(zs_hbm, zs_v)

    def idx_slice(c):
        return idx_all.at[pl.ds(c * _CH, _CH)]

    def ebase(c):
        return pl.multiple_of(wbase + c * _CH, 8)

    def stores_wait(b):
        base0 = pl.multiple_of(wbase, 8)
        pltpu.make_async_copy(fb[b], outf_hbm.at[pl.ds(base0, _CH)],
                              ss[b]).wait()
        pltpu.make_async_copy(db[b], outd_hbm.at[pl.ds(base0, _CH)],
                              ss[b]).wait()

    def body(p, carry):
        for b in range(_RING):
            c = _RING * p + b

            @pl.when(c < _NCH)
            def _():
                @pl.when(c >= _RING)
                def _():
                    stores_wait(b)
                pltpu.async_copy(ft_hbm.at[idx_slice(c)], fb[b], sf[b])

        for b in range(_RING):
            c = _RING * p + b

            @pl.when(c < _NCH)
            def _():
                for g in range(_CH // _L):
                    off = c * _CH + g * _L
                    nbr = idx_all[pl.ds(off, _L)]
                    own = lax.shift_right_logical(
                        ebase0 + wbase + off
                        + jnp.arange(_L, dtype=jnp.int32), 5)
                    dx = (plsc.load_gather(xs_v, [nbr])
                          - plsc.load_gather(xs_v, [own]))
                    dy = (plsc.load_gather(ys_v, [nbr])
                          - plsc.load_gather(ys_v, [own]))
                    dz = (plsc.load_gather(zs_v, [nbr])
                          - plsc.load_gather(zs_v, [own]))
                    db[b][pl.ds(g * _L, _L)] = dx * dx + dy * dy + dz * dz
                pltpu.make_async_copy(ft_hbm.at[idx_slice(c)], fb[b],
                                      sf[b]).wait()
                base = ebase(c)
                pltpu.async_copy(fb[b], outf_hbm.at[pl.ds(base, _CH)], ss[b])
                pltpu.async_copy(db[b], outd_hbm.at[pl.ds(base, _CH)], ss[b])
        return carry

    lax.fori_loop(0, _NPASS, body, 0)
    for b in range(_RING):
        stores_wait(b)


def _sc_gather(ft, xs, ys, zs, idx, ebase0):
    info = plsc.get_sparse_core_info()
    nc = info.num_cores
    mesh = plsc.VectorSubcoreMesh(core_axis_name="c", subcore_axis_name="s")
    fn = functools.partial(
        pl.kernel,
        mesh=mesh,
        out_type=(
            jax.ShapeDtypeStruct((_ES, CIN), jnp.float32),
            jax.ShapeDtypeStruct((_ES,), jnp.float32),
        ),
        scratch_types=(
            [pltpu.VMEM((_EW,), jnp.int32)]
            + [pltpu.VMEM((_CH, CIN), jnp.float32)] * _RING
            + [pltpu.VMEM((_CH,), jnp.float32)] * _RING
            + [pltpu.VMEM((N,), jnp.float32)] * 3
            + [pltpu.SemaphoreType.DMA] * (2 * _RING)
        ),
        compiler_params=pltpu.CompilerParams(needs_layout_passes=False),
    )(functools.partial(_sc_gather_body, nc=nc, ebase0=ebase0))
    return fn(ft, xs, ys, zs, idx)


_NBLK = 200
_GN = 8
_CON = _GN * K
_G = _NBLK // _GN
_ROWS = _GN * NB_BASIS
_DELTA = MAXR / (NB_BASIS - 1)
_C5 = 5.0 * _DELTA


def _tc_body(gf_ref, d2_ref, rm_ref, wbig_ref, out_ref):
    d2 = d2_ref[0]
    d = jnp.minimum(jnp.sqrt(d2 + 1e-12), 6.0)
    rm = rm_ref[0]
    a_lo = jnp.exp(d * d * (-INV2S2)) * rm
    dh = d - _C5
    a_hi = jnp.exp(dh * dh * (-INV2S2)) * rm
    u = jnp.exp(d * (2.0 * INV2S2 * _DELTA))

    rbs = []
    e = a_lo
    rbs.append(e)
    for m in range(1, 5):
        e = e * u
        rbs.append(e * float(np_exp(-INV2S2 * (m * _DELTA) ** 2)))
    e = a_hi
    rbs.append(e)
    for m in range(1, 5):
        e = e * u
        rbs.append(e * float(np_exp(-INV2S2 * _DELTA * _DELTA
                                    * (10 * m + m * m))))

    rbstack = jnp.stack(rbs, axis=1).astype(jnp.bfloat16)
    tiled = jnp.broadcast_to(
        rbstack[:, None, :, :], (_G, _GN, NB_BASIS, _CON)
    ).reshape(_G, _ROWS, _CON)
    ri = lax.broadcasted_iota(jnp.int32, (_ROWS, _CON), 0)
    ci = lax.broadcasted_iota(jnp.int32, (_ROWS, _CON), 1)
    maskf = (ri // NB_BASIS == ci // K).astype(jnp.bfloat16)
    lhs = tiled * maskf[None]

    rhs = gf_ref[...].astype(jnp.bfloat16)
    s = lax.dot_general(
        lhs, rhs,
        dimension_numbers=(((2,), (1,)), ((0,), (0,))),
        preferred_element_type=jnp.float32)
    s_flat = s.reshape(_NBLK, NB_BASIS * CIN).astype(jnp.bfloat16)
    out_ref[...] = lax.dot_general(
        s_flat, wbig_ref[...],
        dimension_numbers=(((1,), (0,)), ((), ())),
        preferred_element_type=jnp.float32)


_NS = N // _NSTAGE


def _tc_compute(gf3, d2g, rmg, wbig):
    grid = (_NS // _NBLK,)
    return pl.pallas_call(
        _tc_body,
        grid=grid,
        in_specs=[
            pl.BlockSpec((_G, _CON, CIN), lambda i: (i, 0, 0)),
            pl.BlockSpec((1, _G, _CON), lambda i: (i, 0, 0)),
            pl.BlockSpec((1, _G, _CON), lambda i: (i, 0, 0)),
            pl.BlockSpec((NB_BASIS * CIN, COUT), lambda i: (0, 0)),
        ],
        out_specs=pl.BlockSpec((_NBLK, COUT), lambda i: (i, 0)),
        out_shape=jax.ShapeDtypeStruct((_NS, COUT), jnp.float32),
        compiler_params=pltpu.CompilerParams(
            dimension_semantics=("arbitrary",)),
    )(gf3, d2g, rmg, wbig)


def kernel(features, geometry, neighbors, rel_mask, W):
    ft = features.T
    xs = geometry[:, 0]
    ys = geometry[:, 1]
    zs = geometry[:, 2]
    idx = neighbors.reshape(-1).astype(jnp.int32)
    wbig = jnp.transpose(W, (0, 2, 1)).reshape(
        NB_BASIS * CIN, COUT).astype(jnp.bfloat16)
    rmg = rel_mask.reshape(N // _NBLK, _G, _CON)
    outs = []
    for s in range(_NSTAGE):
        gf, d2 = _sc_gather(ft, xs, ys, zs,
                            lax.slice(idx, (s * _ES,), ((s + 1) * _ES,)),
                            s * _ES)
        nb0 = s * (_NS // _NBLK)
        outs.append(_tc_compute(
            gf.reshape(_ES // _CON, _CON, CIN),
            d2.reshape(_NS // _NBLK, _G, _CON),
            lax.slice(rmg, (nb0, 0, 0),
                      (nb0 + _NS // _NBLK, _G, _CON)), wbig))
    return jnp.concatenate(outs, axis=0).T


# speedup vs baseline: 1.3001x; 1.0063x over previous
import functools
from math import exp as np_exp

import jax
import jax.numpy as jnp
from jax import lax
from jax.experimental import pallas as pl
from jax.experimental.pallas import tpu as pltpu
from jax.experimental.pallas import tpu_sc as plsc

N = 10000
K = 32
CIN = 128
COUT = 128
NB_BASIS = 10
MAXR = 2.5
SIGMA = MAXR / NB_BASIS
INV2S2 = 1.0 / (2.0 * SIGMA * SIGMA)
E = N * K

_CH = 80
_L = 16


_RING = 4
_NSTAGE = 1
_ES = E // _NSTAGE
_EW = _ES // 32
_NCH = _EW // _CH
_NPASS = (_NCH + _RING - 1) // _RING


def _sc_gather_body(ft_hbm, xs_hbm, ys_hbm, zs_hbm, idx_hbm,
                    outf_hbm, outd_hbm,
                    idx_all, f0, f1, f2, f3, d0, d1, d2b, d3,
                    xs_v, ys_v, zs_v,
                    sf0, sf1, sf2, sf3, ss0, ss1, ss2, ss3, nc, ebase0):
    wid = lax.axis_index("s") * nc + lax.axis_index("c")
    wbase = wid * _EW
    fb = [f0, f1, f2, f3]
    db = [d0, d1, d2b, d3]
    sf = [sf0, sf1, sf2, sf3]
    ss = [ss0, ss1, ss2, ss3]

    pltpu.sync_copy(idx_hbm.at[pl.ds(pl.multiple_of(wbase, 8), _EW)], idx_all)
    pltpu.sync_copy(xs_hbm, xs_v)
    pltpu.sync_copy(ys_hbm, ys_v)
    pltpu.sync_copy(zs_hbm, zs_v)

    def idx_slice(c):
        return idx_all.at[pl.ds(c * _CH, _CH)]

    def ebase(c):
        return pl.multiple_of(wbase + c * _CH, 8)

    def stores_wait(b):
        base0 = pl.multiple_of(wbase, 8)
        pltpu.make_async_copy(fb[b], outf_hbm.at[pl.ds(base0, _CH)],
                              ss[b]).wait()
        pltpu.make_async_copy(db[b], outd_hbm.at[pl.ds(base0, _CH)],
                              ss[b]).wait()

    def body(p, carry):
        for b in range(_RING):
            c = _RING * p + b

            @pl.when(c < _NCH)
            def _():
                @pl.when(c >= _RING)
                def _():
                    stores_wait(b)
                pltpu.async_copy(ft_hbm.at[idx_slice(c)], fb[b], sf[b])

        for b in range(_RING):
            c = _RING * p + b

            @pl.when(c < _NCH)
            def _():
                for g in range(_CH // _L):
                    off = c * _CH + g * _L
                    nbr = idx_all[pl.ds(off, _L)]
                    own = lax.shift_right_logical(
                        ebase0 + wbase + off
                        + jnp.arange(_L, dtype=jnp.int32), 5)
                    dx = (plsc.load_gather(xs_v, [nbr])
                          - plsc.load_gather(xs_v, [own]))
                    dy = (plsc.load_gather(ys_v, [nbr])
                          - plsc.load_gather(ys_v, [own]))
                    dz = (plsc.load_gather(zs_v, [nbr])
                          - plsc.load_gather(zs_v, [own]))
                    db[b][pl.ds(g * _L, _L)] = dx * dx + dy * dy + dz * dz
                pltpu.make_async_copy(ft_hbm.at[idx_slice(c)], fb[b],
                                      sf[b]).wait()
                base = ebase(c)
                pltpu.async_copy(fb[b], outf_hbm.at[pl.ds(base, _CH)], ss[b])
                pltpu.async_copy(db[b], outd_hbm.at[pl.ds(base, _CH)], ss[b])
        return carry

    lax.fori_loop(0, _NPASS, body, 0)
    for b in range(_RING):
        stores_wait(b)


def _sc_gather(ft, xs, ys, zs, idx, ebase0):
    info = plsc.get_sparse_core_info()
    nc = info.num_cores
    mesh = plsc.VectorSubcoreMesh(core_axis_name="c", subcore_axis_name="s")
    fn = functools.partial(
        pl.kernel,
        mesh=mesh,
        out_type=(
            jax.ShapeDtypeStruct((_ES, CIN), jnp.float32),
            jax.ShapeDtypeStruct((_ES,), jnp.float32),
        ),
        scratch_types=(
            [pltpu.VMEM((_EW,), jnp.int32)]
            + [pltpu.VMEM((_CH, CIN), jnp.float32)] * _RING
            + [pltpu.VMEM((_CH,), jnp.float32)] * _RING
            + [pltpu.VMEM((N,), jnp.float32)] * 3
            + [pltpu.SemaphoreType.DMA] * (2 * _RING)
        ),
        compiler_params=pltpu.CompilerParams(needs_layout_passes=False),
    )(functools.partial(_sc_gather_body, nc=nc, ebase0=ebase0))
    return fn(ft, xs, ys, zs, idx)


_NBLK = 200
_GN = 8
_CON = _GN * K
_G = _NBLK // _GN
_ROWS = _GN * NB_BASIS
_DELTA = MAXR / (NB_BASIS - 1)
_C5 = 5.0 * _DELTA


def _tc_body(gf_ref, d2_ref, rm_ref, wbig_ref, out_ref):
    d2 = d2_ref[0]
    d = jnp.minimum(jnp.sqrt(d2 + 1e-12), 6.0)
    rm = rm_ref[0]
    a_lo = jnp.exp(d * d * (-INV2S2)) * rm
    dh = d - _C5
    a_hi = jnp.exp(dh * dh * (-INV2S2)) * rm
    u = jnp.exp(d * (2.0 * INV2S2 * _DELTA))

    rbs = []
    e = a_lo
    rbs.append(e)
    for m in range(1, 5):
        e = e * u
        rbs.append(e * float(np_exp(-INV2S2 * (m * _DELTA) ** 2)))
    e = a_hi
    rbs.append(e)
    for m in range(1, 5):
        e = e * u
        rbs.append(e * float(np_exp(-INV2S2 * _DELTA * _DELTA
                                    * (10 * m + m * m))))

    rbstack = jnp.stack(rbs, axis=1)
    tiled = jnp.broadcast_to(
        rbstack[:, None, :, :], (_G, _GN, NB_BASIS, _CON)
    ).reshape(_G, _ROWS, _CON)
    ri = lax.broadcasted_iota(jnp.int32, (_ROWS, _CON), 0)
    ci = lax.broadcasted_iota(jnp.int32, (_ROWS, _CON), 1)
    maskf = (ri // NB_BASIS == ci // K).astype(jnp.float32)
    lhs = (tiled * maskf[None]).astype(jnp.bfloat16)

    rhs = gf_ref[...].astype(jnp.bfloat16)
    s = lax.dot_general(
        lhs, rhs,
        dimension_numbers=(((2,), (1,)), ((0,), (0,))),
        preferred_element_type=jnp.float32)
    s_flat = s.reshape(_NBLK, NB_BASIS * CIN).astype(jnp.bfloat16)
    out_ref[...] = lax.dot_general(
        s_flat, wbig_ref[...],
        dimension_numbers=(((1,), (0,)), ((), ())),
        preferred_element_type=jnp.float32)


_NS = N // _NSTAGE


def _tc_compute(gf3, d2g, rmg, wbig):
    grid = (_NS // _NBLK,)
    return pl.pallas_call(
        _tc_body,
        grid=grid,
        in_specs=[
            pl.BlockSpec((_G, _CON, CIN), lambda i: (i, 0, 0)),
            pl.BlockSpec((1, _G, _CON), lambda i: (i, 0, 0)),
            pl.BlockSpec((1, _G, _CON), lambda i: (i, 0, 0)),
            pl.BlockSpec((NB_BASIS * CIN, COUT), lambda i: (0, 0)),
        ],
        out_specs=pl.BlockSpec((_NBLK, COUT), lambda i: (i, 0)),
        out_shape=jax.ShapeDtypeStruct((_NS, COUT), jnp.float32),
        compiler_params=pltpu.CompilerParams(
            dimension_semantics=("arbitrary",)),
    )(gf3, d2g, rmg, wbig)


def kernel(features, geometry, neighbors, rel_mask, W):
    ft = features.T
    xs = geometry[:, 0]
    ys = geometry[:, 1]
    zs = geometry[:, 2]
    idx = neighbors.reshape(-1).astype(jnp.int32)
    wbig = jnp.transpose(W, (0, 2, 1)).reshape(
        NB_BASIS * CIN, COUT).astype(jnp.bfloat16)
    rmg = rel_mask.reshape(N // _NBLK, _G, _CON)
    outs = []
    for s in range(_NSTAGE):
        gf, d2 = _sc_gather(ft, xs, ys, zs,
                            lax.slice(idx, (s * _ES,), ((s + 1) * _ES,)),
                            s * _ES)
        nb0 = s * (_NS // _NBLK)
        outs.append(_tc_compute(
            gf.reshape(_ES // _CON, _CON, CIN),
            d2.reshape(_NS // _NBLK, _G, _CON),
            lax.slice(rmg, (nb0, 0, 0),
                      (nb0 + _NS // _NBLK, _G, _CON)), wbig))
    return jnp.concatenate(outs, axis=0).T
